# Initial kernel scaffold; baseline (speedup 1.0000x reference)
import jax
import jax.numpy as jnp
from jax.experimental import pallas as pl


def kernel(features, edge_index, W_self1, W_neigh1, b1, W_self2, W_neigh2, b2, W_self3, W_neigh3, b3):
    raise NotImplementedError("write your pallas kernel here")


# speedup vs baseline: 1.6972x; 1.6972x over previous
import functools
import jax
import jax.numpy as jnp
from jax import lax
from jax.experimental import pallas as pl
from jax.experimental.pallas import tpu as pltpu
from jax.experimental.pallas import tpu_sc as plsc

N_NODES = 10000
NP = 10240
HALF = NP // 2
SPN = 6144
D = 128
DA = D + 16
E = 320000
CHUNK = 64
NC = 2
NS = 16
T_PER_TILE = 320
GRP = 8
NROW2D = NS * T_PER_TILE
EP = NROW2D * CHUNK
ZBLK = SPN // (CHUNK * NS)


def _sc_agg_body(h_hbm, src_hbm, dst_hbm, zrows_hbm, agg_out,
                 src_idx, dst_idx, rows0, rows1, agg_sp, sem0, sem1):
    c = lax.axis_index("c")
    s = lax.axis_index("s")

    pltpu.sync_copy(zrows_hbm, rows0)

    def zbody(j, carry):
        pltpu.sync_copy(rows0,
                        agg_sp.at[pl.ds((s * ZBLK + j) * CHUNK, CHUNK)])
        return carry

    lax.fori_loop(0, ZBLK, zbody, 0)

    plsc.subcore_barrier()

    def body(g, carry):
        base = s * T_PER_TILE + g * GRP
        pltpu.sync_copy(src_hbm.at[pl.ds(base, GRP)], src_idx)
        pltpu.sync_copy(dst_hbm.at[c, pl.ds(base, GRP)], dst_idx)
        for k in range(0, GRP, 2):
            cp0 = pltpu.async_copy(h_hbm.at[src_idx.at[k]], rows0, sem0)
            cp1 = pltpu.async_copy(h_hbm.at[src_idx.at[k + 1]], rows1, sem1)
            cp0.wait()
            pltpu.sync_copy(rows0, agg_sp.at[dst_idx.at[k]], add=True)
            cp1.wait()
            pltpu.sync_copy(rows1, agg_sp.at[dst_idx.at[k + 1]], add=True)
        return carry

    lax.fori_loop(0, T_PER_TILE // GRP, body, 0)

    plsc.subcore_barrier()

    def obody(j, carry):
        rr = (s * ZBLK + j) * CHUNK
        pltpu.sync_copy(agg_sp.at[pl.ds(rr, CHUNK)], rows0)
        pltpu.sync_copy(rows0, agg_out.at[c, pl.ds(rr, CHUNK)])
        return carry

    lax.fori_loop(0, ZBLK, obody, 0)


@functools.cache
def _get_sc_agg():
  mesh = plsc.VectorSubcoreMesh(core_axis_name="c", subcore_axis_name="s")
  return pl.kernel(
    _sc_agg_body,
    mesh=mesh,
    out_type=[
        jax.ShapeDtypeStruct((NC, SPN, DA), jnp.float32),
    ],
    scratch_types=[
        pltpu.VMEM((GRP, CHUNK), jnp.int32),
        pltpu.VMEM((GRP, CHUNK), jnp.int32),
        pltpu.VMEM((CHUNK, DA), jnp.float32),
        pltpu.VMEM((CHUNK, DA), jnp.float32),
        pltpu.VMEM_SHARED((SPN, DA), jnp.float32),
        pltpu.SemaphoreType.DMA,
        pltpu.SemaphoreType.DMA,
    ],
    compiler_params=pltpu.CompilerParams(use_tc_tiling_on_sc=False),
  )


def _tc_layer_body(h_ref, hn_ref, deg_ref, ws_ref, wn_ref, b_ref, o_ref,
                   *, relu):
    rdeg = 1.0 / jnp.maximum(deg_ref[:, 0:1], 1.0)
    hn = hn_ref[:, :] * rdeg
    out = (jnp.dot(h_ref[:, :], ws_ref[:, :],
                   preferred_element_type=jnp.float32)
           + jnp.dot(hn, wn_ref[:, :], preferred_element_type=jnp.float32)
           + b_ref[:, :])
    if relu:
        out = jnp.maximum(out, 0.0)
    o_ref[:, :] = out


_BR = 1024


def _tc_layer(h, hn, degv, ws, wn, b, relu):
    return pl.pallas_call(
        functools.partial(_tc_layer_body, relu=relu),
        grid=(NP // _BR,),
        in_specs=[
            pl.BlockSpec((_BR, D), lambda i: (i, 0)),
            pl.BlockSpec((_BR, D), lambda i: (i, 0)),
            pl.BlockSpec((_BR, 8), lambda i: (i, 0)),
            pl.BlockSpec((D, D), lambda i: (0, 0)),
            pl.BlockSpec((D, D), lambda i: (0, 0)),
            pl.BlockSpec((1, D), lambda i: (0, 0)),
        ],
        out_specs=pl.BlockSpec((_BR, D), lambda i: (i, 0)),
        out_shape=jax.ShapeDtypeStruct((NP, D), jnp.float32),
    )(h, hn, degv, ws, wn, b)


def _aggregate(sc_agg, h, src2d, dstr, zrows):
    h_aug = jnp.concatenate([h, jnp.ones((NP, DA - D), jnp.float32)], axis=1)
    [agg] = sc_agg(h_aug, src2d, dstr, zrows)
    agg_full = jnp.concatenate([agg[0, :HALF], agg[1, :HALF]], axis=0)
    return agg_full[:, :D], agg_full[:, D:D + 8]


def kernel(features, edge_index, W_self1, W_neigh1, b1,
           W_self2, W_neigh2, b2, W_self3, W_neigh3, b3):
    src = edge_index[0]
    dst = edge_index[1]
    pad_e = EP - E
    src_p = jnp.concatenate([src, jnp.zeros((pad_e,), jnp.int32)])
    dst_p = jnp.concatenate([dst, jnp.full((pad_e,), N_NODES, jnp.int32)])
    src2d = src_p.reshape(NROW2D, CHUNK)
    dst0 = jnp.where(dst_p < HALF, dst_p, HALF)
    dst1 = jnp.where(dst_p >= HALF, dst_p - HALF, HALF)
    dstr = jnp.stack([dst0.reshape(NROW2D, CHUNK),
                      dst1.reshape(NROW2D, CHUNK)])

    h0 = jnp.pad(features, ((0, NP - N_NODES), (0, 0)))
    zrows = jnp.zeros((CHUNK, DA), jnp.float32)

    w3 = jnp.pad(W_self3, ((0, 0), (0, D - W_self3.shape[1])))
    wn3 = jnp.pad(W_neigh3, ((0, 0), (0, D - W_neigh3.shape[1])))
    b3p = jnp.pad(b3, (0, D - b3.shape[0]))

    sc_agg = _get_sc_agg()
    hn, degv = _aggregate(sc_agg, h0, src2d, dstr, zrows)
    h1 = _tc_layer(h0, hn, degv, W_self1, W_neigh1, b1.reshape(1, D),
                   relu=True)
    hn, degv = _aggregate(sc_agg, h1, src2d, dstr, zrows)
    h2 = _tc_layer(h1, hn, degv, W_self2, W_neigh2, b2.reshape(1, D),
                   relu=True)
    hn, degv = _aggregate(sc_agg, h2, src2d, dstr, zrows)
    h3 = _tc_layer(h2, hn, degv, w3, wn3, b3p.reshape(1, D), relu=False)
    return h3[:N_NODES, :W_self3.shape[1]]


# speedup vs baseline: 1.7078x; 1.0062x over previous
import functools
import jax
import jax.numpy as jnp
from jax import lax
from jax.experimental import pallas as pl
from jax.experimental.pallas import tpu as pltpu
from jax.experimental.pallas import tpu_sc as plsc

N_NODES = 10000
NP = 10240
HALF = NP // 2
SPN = 6144
D = 128
DA = D + 16
E = 320000
CHUNK = 64
NC = 2
NS = 16
T_PER_TILE = 320
GRP = 8
NROW2D = NS * T_PER_TILE
EP = NROW2D * CHUNK
ZBLK = SPN // (CHUNK * NS)


def _sc_agg_body(h_hbm, src_hbm, dst_hbm, zrows_hbm, agg_out,
                 src_idx, dst_idx, rows0, rows1, agg_sp,
                 sem0, sem1, sem_s0, sem_s1):
    c = lax.axis_index("c")
    s = lax.axis_index("s")

    pltpu.sync_copy(zrows_hbm, rows0)

    def zbody(j, carry):
        pltpu.sync_copy(rows0,
                        agg_sp.at[pl.ds((s * ZBLK + j) * CHUNK, CHUNK)])
        return carry

    lax.fori_loop(0, ZBLK, zbody, 0)

    plsc.subcore_barrier()

    def _drain_scatters(k0, k1):
        pltpu.make_async_copy(rows0, agg_sp.at[dst_idx.at[k0]],
                              sem_s0).wait()
        pltpu.make_async_copy(rows1, agg_sp.at[dst_idx.at[k1]],
                              sem_s1).wait()

    def body(g, carry):
        @pl.when(g > 0)
        def _():
            _drain_scatters(GRP - 2, GRP - 1)

        base = s * T_PER_TILE + g * GRP
        pltpu.sync_copy(src_hbm.at[pl.ds(base, GRP)], src_idx)
        pltpu.sync_copy(dst_hbm.at[c, pl.ds(base, GRP)], dst_idx)
        for k in range(0, GRP, 2):
            if k > 0:
                _drain_scatters(k - 2, k - 1)
            cp0 = pltpu.async_copy(h_hbm.at[src_idx.at[k]], rows0, sem0)
            cp1 = pltpu.async_copy(h_hbm.at[src_idx.at[k + 1]], rows1, sem1)
            cp0.wait()
            pltpu.async_copy(rows0, agg_sp.at[dst_idx.at[k]], sem_s0,
                             add=True)
            cp1.wait()
            pltpu.async_copy(rows1, agg_sp.at[dst_idx.at[k + 1]], sem_s1,
                             add=True)
        return carry

    lax.fori_loop(0, T_PER_TILE // GRP, body, 0)
    _drain_scatters(GRP - 2, GRP - 1)

    plsc.subcore_barrier()

    def obody(j, carry):
        rr = (s * ZBLK + j) * CHUNK
        pltpu.sync_copy(agg_sp.at[pl.ds(rr, CHUNK)], rows0)
        pltpu.sync_copy(rows0, agg_out.at[c, pl.ds(rr, CHUNK)])
        return carry

    lax.fori_loop(0, ZBLK, obody, 0)


@functools.cache
def _get_sc_agg():
  mesh = plsc.VectorSubcoreMesh(core_axis_name="c", subcore_axis_name="s")
  return pl.kernel(
    _sc_agg_body,
    mesh=mesh,
    out_type=[
        jax.ShapeDtypeStruct((NC, SPN, DA), jnp.float32),
    ],
    scratch_types=[
        pltpu.VMEM((GRP, CHUNK), jnp.int32),
        pltpu.VMEM((GRP, CHUNK), jnp.int32),
        pltpu.VMEM((CHUNK, DA), jnp.float32),
        pltpu.VMEM((CHUNK, DA), jnp.float32),
        pltpu.VMEM_SHARED((SPN, DA), jnp.float32),
        pltpu.SemaphoreType.DMA,
        pltpu.SemaphoreType.DMA,
        pltpu.SemaphoreType.DMA,
        pltpu.SemaphoreType.DMA,
    ],
    compiler_params=pltpu.CompilerParams(use_tc_tiling_on_sc=False),
  )


def _tc_layer_body(h_ref, hn_ref, deg_ref, ws_ref, wn_ref, b_ref, o_ref,
                   *, relu):
    rdeg = 1.0 / jnp.maximum(deg_ref[:, 0:1], 1.0)
    hn = hn_ref[:, :] * rdeg
    out = (jnp.dot(h_ref[:, :], ws_ref[:, :],
                   preferred_element_type=jnp.float32)
           + jnp.dot(hn, wn_ref[:, :], preferred_element_type=jnp.float32)
           + b_ref[:, :])
    if relu:
        out = jnp.maximum(out, 0.0)
    o_ref[:, :] = out


_BR = 1024


def _tc_layer(h, hn, degv, ws, wn, b, relu):
    return pl.pallas_call(
        functools.partial(_tc_layer_body, relu=relu),
        grid=(NP // _BR,),
        in_specs=[
            pl.BlockSpec((_BR, D), lambda i: (i, 0)),
            pl.BlockSpec((_BR, D), lambda i: (i, 0)),
            pl.BlockSpec((_BR, 8), lambda i: (i, 0)),
            pl.BlockSpec((D, D), lambda i: (0, 0)),
            pl.BlockSpec((D, D), lambda i: (0, 0)),
            pl.BlockSpec((1, D), lambda i: (0, 0)),
        ],
        out_specs=pl.BlockSpec((_BR, D), lambda i: (i, 0)),
        out_shape=jax.ShapeDtypeStruct((NP, D), jnp.float32),
    )(h, hn, degv, ws, wn, b)


def _aggregate(sc_agg, h, src2d, dstr, zrows):
    h_aug = jnp.concatenate([h, jnp.ones((NP, DA - D), jnp.float32)], axis=1)
    [agg] = sc_agg(h_aug, src2d, dstr, zrows)
    agg_full = jnp.concatenate([agg[0, :HALF], agg[1, :HALF]], axis=0)
    return agg_full[:, :D], agg_full[:, D:D + 8]


def kernel(features, edge_index, W_self1, W_neigh1, b1,
           W_self2, W_neigh2, b2, W_self3, W_neigh3, b3):
    src = edge_index[0]
    dst = edge_index[1]
    pad_e = EP - E
    src_p = jnp.concatenate([src, jnp.zeros((pad_e,), jnp.int32)])
    dst_p = jnp.concatenate([dst, jnp.full((pad_e,), N_NODES, jnp.int32)])
    src2d = src_p.reshape(NROW2D, CHUNK)
    dst0 = jnp.where(dst_p < HALF, dst_p, HALF)
    dst1 = jnp.where(dst_p >= HALF, dst_p - HALF, HALF)
    dstr = jnp.stack([dst0.reshape(NROW2D, CHUNK),
                      dst1.reshape(NROW2D, CHUNK)])

    h0 = jnp.pad(features, ((0, NP - N_NODES), (0, 0)))
    zrows = jnp.zeros((CHUNK, DA), jnp.float32)

    w3 = jnp.pad(W_self3, ((0, 0), (0, D - W_self3.shape[1])))
    wn3 = jnp.pad(W_neigh3, ((0, 0), (0, D - W_neigh3.shape[1])))
    b3p = jnp.pad(b3, (0, D - b3.shape[0]))

    sc_agg = _get_sc_agg()
    hn, degv = _aggregate(sc_agg, h0, src2d, dstr, zrows)
    h1 = _tc_layer(h0, hn, degv, W_self1, W_neigh1, b1.reshape(1, D),
                   relu=True)
    hn, degv = _aggregate(sc_agg, h1, src2d, dstr, zrows)
    h2 = _tc_layer(h1, hn, degv, W_self2, W_neigh2, b2.reshape(1, D),
                   relu=True)
    hn, degv = _aggregate(sc_agg, h2, src2d, dstr, zrows)
    h3 = _tc_layer(h2, hn, degv, w3, wn3, b3p.reshape(1, D), relu=False)
    return h3[:N_NODES, :W_self3.shape[1]]


# speedup vs baseline: 1.7284x; 1.0121x over previous
import functools
import jax
import jax.numpy as jnp
from jax import lax
from jax.experimental import pallas as pl
from jax.experimental.pallas import tpu as pltpu
from jax.experimental.pallas import tpu_sc as plsc

N_NODES = 10000
NP = 10240
HALF = NP // 2
SPN = 6144
D = 128
DA = D + 16
E = 320000
CHUNK = 128
NC = 2
NS = 16
T_PER_TILE = 160
GRP = 4
NROW2D = NS * T_PER_TILE
EP = NROW2D * CHUNK
ZBLK = SPN // (CHUNK * NS)


def _sc_agg_body(h_hbm, src_hbm, dst_hbm, zrows_hbm, agg_out,
                 src_idx, dst_idx, rows0, rows1, agg_sp,
                 sem0, sem1, sem_s0, sem_s1):
    c = lax.axis_index("c")
    s = lax.axis_index("s")

    pltpu.sync_copy(zrows_hbm, rows0)

    def zbody(j, carry):
        pltpu.sync_copy(rows0,
                        agg_sp.at[pl.ds((s * ZBLK + j) * CHUNK, CHUNK)])
        return carry

    lax.fori_loop(0, ZBLK, zbody, 0)

    plsc.subcore_barrier()

    def _drain_scatters(k0, k1):
        pltpu.make_async_copy(rows0, agg_sp.at[dst_idx.at[k0]],
                              sem_s0).wait()
        pltpu.make_async_copy(rows1, agg_sp.at[dst_idx.at[k1]],
                              sem_s1).wait()

    def body(g, carry):
        @pl.when(g > 0)
        def _():
            _drain_scatters(GRP - 2, GRP - 1)

        base = s * T_PER_TILE + g * GRP
        pltpu.sync_copy(src_hbm.at[pl.ds(base, GRP)], src_idx)
        pltpu.sync_copy(dst_hbm.at[c, pl.ds(base, GRP)], dst_idx)
        for k in range(0, GRP, 2):
            if k > 0:
                _drain_scatters(k - 2, k - 1)
            cp0 = pltpu.async_copy(h_hbm.at[src_idx.at[k]], rows0, sem0)
            cp1 = pltpu.async_copy(h_hbm.at[src_idx.at[k + 1]], rows1, sem1)
            cp0.wait()
            pltpu.async_copy(rows0, agg_sp.at[dst_idx.at[k]], sem_s0,
                             add=True)
            cp1.wait()
            pltpu.async_copy(rows1, agg_sp.at[dst_idx.at[k + 1]], sem_s1,
                             add=True)
        return carry

    lax.fori_loop(0, T_PER_TILE // GRP, body, 0)
    _drain_scatters(GRP - 2, GRP - 1)

    plsc.subcore_barrier()

    def obody(j, carry):
        rr = (s * ZBLK + j) * CHUNK
        pltpu.sync_copy(agg_sp.at[pl.ds(rr, CHUNK)], rows0)
        pltpu.sync_copy(rows0, agg_out.at[c, pl.ds(rr, CHUNK)])
        return carry

    lax.fori_loop(0, ZBLK, obody, 0)


@functools.cache
def _get_sc_agg():
  mesh = plsc.VectorSubcoreMesh(core_axis_name="c", subcore_axis_name="s")
  return pl.kernel(
    _sc_agg_body,
    mesh=mesh,
    out_type=[
        jax.ShapeDtypeStruct((NC, SPN, DA), jnp.float32),
    ],
    scratch_types=[
        pltpu.VMEM((GRP, CHUNK), jnp.int32),
        pltpu.VMEM((GRP, CHUNK), jnp.int32),
        pltpu.VMEM((CHUNK, DA), jnp.float32),
        pltpu.VMEM((CHUNK, DA), jnp.float32),
        pltpu.VMEM_SHARED((SPN, DA), jnp.float32),
        pltpu.SemaphoreType.DMA,
        pltpu.SemaphoreType.DMA,
        pltpu.SemaphoreType.DMA,
        pltpu.SemaphoreType.DMA,
    ],
    compiler_params=pltpu.CompilerParams(use_tc_tiling_on_sc=False),
  )


def _tc_layer_body(h_ref, hn_ref, deg_ref, ws_ref, wn_ref, b_ref, o_ref,
                   *, relu):
    rdeg = 1.0 / jnp.maximum(deg_ref[:, 0:1], 1.0)
    hn = hn_ref[:, :] * rdeg
    out = (jnp.dot(h_ref[:, :], ws_ref[:, :],
                   preferred_element_type=jnp.float32)
           + jnp.dot(hn, wn_ref[:, :], preferred_element_type=jnp.float32)
           + b_ref[:, :])
    if relu:
        out = jnp.maximum(out, 0.0)
    o_ref[:, :] = out


_BR = 1024


def _tc_layer(h, hn, degv, ws, wn, b, relu):
    return pl.pallas_call(
        functools.partial(_tc_layer_body, relu=relu),
        grid=(NP // _BR,),
        in_specs=[
            pl.BlockSpec((_BR, D), lambda i: (i, 0)),
            pl.BlockSpec((_BR, D), lambda i: (i, 0)),
            pl.BlockSpec((_BR, 8), lambda i: (i, 0)),
            pl.BlockSpec((D, D), lambda i: (0, 0)),
            pl.BlockSpec((D, D), lambda i: (0, 0)),
            pl.BlockSpec((1, D), lambda i: (0, 0)),
        ],
        out_specs=pl.BlockSpec((_BR, D), lambda i: (i, 0)),
        out_shape=jax.ShapeDtypeStruct((NP, D), jnp.float32),
    )(h, hn, degv, ws, wn, b)


def _aggregate(sc_agg, h, src2d, dstr, zrows):
    h_aug = jnp.concatenate([h, jnp.ones((NP, DA - D), jnp.float32)], axis=1)
    [agg] = sc_agg(h_aug, src2d, dstr, zrows)
    agg_full = jnp.concatenate([agg[0, :HALF], agg[1, :HALF]], axis=0)
    return agg_full[:, :D], agg_full[:, D:D + 8]


def kernel(features, edge_index, W_self1, W_neigh1, b1,
           W_self2, W_neigh2, b2, W_self3, W_neigh3, b3):
    src = edge_index[0]
    dst = edge_index[1]
    pad_e = EP - E
    src_p = jnp.concatenate([src, jnp.zeros((pad_e,), jnp.int32)])
    dst_p = jnp.concatenate([dst, jnp.full((pad_e,), N_NODES, jnp.int32)])
    src2d = src_p.reshape(NROW2D, CHUNK)
    dst0 = jnp.where(dst_p < HALF, dst_p, HALF)
    dst1 = jnp.where(dst_p >= HALF, dst_p - HALF, HALF)
    dstr = jnp.stack([dst0.reshape(NROW2D, CHUNK),
                      dst1.reshape(NROW2D, CHUNK)])

    h0 = jnp.pad(features, ((0, NP - N_NODES), (0, 0)))
    zrows = jnp.zeros((CHUNK, DA), jnp.float32)

    w3 = jnp.pad(W_self3, ((0, 0), (0, D - W_self3.shape[1])))
    wn3 = jnp.pad(W_neigh3, ((0, 0), (0, D - W_neigh3.shape[1])))
    b3p = jnp.pad(b3, (0, D - b3.shape[0]))

    sc_agg = _get_sc_agg()
    hn, degv = _aggregate(sc_agg, h0, src2d, dstr, zrows)
    h1 = _tc_layer(h0, hn, degv, W_self1, W_neigh1, b1.reshape(1, D),
                   relu=True)
    hn, degv = _aggregate(sc_agg, h1, src2d, dstr, zrows)
    h2 = _tc_layer(h1, hn, degv, W_self2, W_neigh2, b2.reshape(1, D),
                   relu=True)
    hn, degv = _aggregate(sc_agg, h2, src2d, dstr, zrows)
    h3 = _tc_layer(h2, hn, degv, w3, wn3, b3p.reshape(1, D), relu=False)
    return h3[:N_NODES, :W_self3.shape[1]]


# speedup vs baseline: 1.7616x; 1.0192x over previous
import functools
import jax
import jax.numpy as jnp
from jax import lax
from jax.experimental import pallas as pl
from jax.experimental.pallas import tpu as pltpu
from jax.experimental.pallas import tpu_sc as plsc

N_NODES = 10000
NP = 10240
HALF = NP // 2
SPN = 6144
D = 128
DA = D + 16
E = 320000
CHUNK = 128
NC = 2
NS = 16
T_PER_TILE = 160
GRP = 4
NROW2D = NS * T_PER_TILE
EP = NROW2D * CHUNK
ZBLK = SPN // (CHUNK * NS)


def _sc_agg_body(h_hbm, src_hbm, dst_hbm, zrows_hbm, agg_out,
                 src_idx, dst_idx, rows0, rows1, agg_sp,
                 sem0, sem1, sem_s0, sem_s1):
    c = lax.axis_index("c")
    s = lax.axis_index("s")

    pltpu.sync_copy(zrows_hbm, rows0)

    def zbody(j, carry):
        pltpu.sync_copy(rows0,
                        agg_sp.at[pl.ds((s * ZBLK + j) * CHUNK, CHUNK)])
        return carry

    lax.fori_loop(0, ZBLK, zbody, 0)

    plsc.subcore_barrier()

    def _drain_scatters(k0, k1):
        pltpu.make_async_copy(rows0, agg_sp.at[dst_idx.at[k0]],
                              sem_s0).wait()
        pltpu.make_async_copy(rows1, agg_sp.at[dst_idx.at[k1]],
                              sem_s1).wait()

    def body(g, carry):
        @pl.when(g > 0)
        def _():
            _drain_scatters(GRP - 2, GRP - 1)

        base = s * T_PER_TILE + g * GRP
        pltpu.sync_copy(src_hbm.at[pl.ds(base, GRP)], src_idx)
        pltpu.sync_copy(dst_hbm.at[c, pl.ds(base, GRP)], dst_idx)
        for k in range(0, GRP, 2):
            if k > 0:
                _drain_scatters(k - 2, k - 1)
            cp0 = pltpu.async_copy(h_hbm.at[src_idx.at[k]], rows0, sem0)
            cp1 = pltpu.async_copy(h_hbm.at[src_idx.at[k + 1]], rows1, sem1)
            cp0.wait()
            pltpu.async_copy(rows0, agg_sp.at[dst_idx.at[k]], sem_s0,
                             add=True)
            cp1.wait()
            pltpu.async_copy(rows1, agg_sp.at[dst_idx.at[k + 1]], sem_s1,
                             add=True)
        return carry

    lax.fori_loop(0, T_PER_TILE // GRP, body, 0)
    _drain_scatters(GRP - 2, GRP - 1)

    plsc.subcore_barrier()

    def obody(j, carry):
        rr = (s * ZBLK + j) * CHUNK
        pltpu.sync_copy(agg_sp.at[pl.ds(rr, CHUNK)], rows0)
        pltpu.sync_copy(rows0, agg_out.at[c, pl.ds(rr, CHUNK)])
        return carry

    lax.fori_loop(0, ZBLK, obody, 0)


@functools.cache
def _get_sc_agg():
  mesh = plsc.VectorSubcoreMesh(core_axis_name="c", subcore_axis_name="s")
  return pl.kernel(
    _sc_agg_body,
    mesh=mesh,
    out_type=[
        jax.ShapeDtypeStruct((NC, SPN, DA), jnp.float32),
    ],
    scratch_types=[
        pltpu.VMEM((GRP, CHUNK), jnp.int32),
        pltpu.VMEM((GRP, CHUNK), jnp.int32),
        pltpu.VMEM((CHUNK, DA), jnp.float32),
        pltpu.VMEM((CHUNK, DA), jnp.float32),
        pltpu.VMEM_SHARED((SPN, DA), jnp.float32),
        pltpu.SemaphoreType.DMA,
        pltpu.SemaphoreType.DMA,
        pltpu.SemaphoreType.DMA,
        pltpu.SemaphoreType.DMA,
    ],
    compiler_params=pltpu.CompilerParams(use_tc_tiling_on_sc=False),
  )


def _tc_layer_body(h_ref, hn_ref, deg_ref, ws_ref, wn_ref, b_ref, o_ref,
                   *, relu):
    rdeg = 1.0 / jnp.maximum(deg_ref[:, 0:1], 1.0)
    hn = hn_ref[:, :] * rdeg
    out = (jnp.dot(h_ref[:, :], ws_ref[:, :],
                   preferred_element_type=jnp.float32)
           + jnp.dot(hn, wn_ref[:, :], preferred_element_type=jnp.float32)
           + b_ref[:, :])
    if relu:
        out = jnp.maximum(out, 0.0)
    o_ref[:, :] = out


_BR = 1024


def _tc_layer(h, hn, degv, ws, wn, b, relu):
    return pl.pallas_call(
        functools.partial(_tc_layer_body, relu=relu),
        grid=(NP // _BR,),
        in_specs=[
            pl.BlockSpec((_BR, D), lambda i: (i, 0)),
            pl.BlockSpec((_BR, D), lambda i: (i, 0)),
            pl.BlockSpec((_BR, 8), lambda i: (i, 0)),
            pl.BlockSpec((D, D), lambda i: (0, 0)),
            pl.BlockSpec((D, D), lambda i: (0, 0)),
            pl.BlockSpec((1, D), lambda i: (0, 0)),
        ],
        out_specs=pl.BlockSpec((_BR, D), lambda i: (i, 0)),
        out_shape=jax.ShapeDtypeStruct((NP, D), jnp.float32),
    )(h, hn, degv, ws, wn, b)


def _aggregate(sc_agg, h, src2d, dstr, zrows):
    h_aug = jnp.concatenate([h, jnp.ones((NP, DA - D), jnp.float32)], axis=1)
    [agg] = sc_agg(h_aug, src2d, dstr, zrows)
    agg_full = jnp.concatenate([agg[0, :HALF], agg[1, :HALF]], axis=0)
    return agg_full[:, :D], agg_full[:, D:D + 8]


def kernel(features, edge_index, W_self1, W_neigh1, b1,
           W_self2, W_neigh2, b2, W_self3, W_neigh3, b3):
    src = edge_index[0]
    dst = edge_index[1]
    pad_e = EP - E
    src_p = jnp.concatenate([src, jnp.zeros((pad_e,), jnp.int32)])
    dst_p = jnp.concatenate([dst, jnp.full((pad_e,), N_NODES, jnp.int32)])
    src2d = src_p.reshape(NROW2D, CHUNK)
    sink = HALF + (jnp.arange(EP, dtype=jnp.int32) & (SPN - HALF - 1))
    dst0 = jnp.where(dst_p < HALF, dst_p, sink)
    dst1 = jnp.where(dst_p >= HALF, dst_p - HALF, sink)
    dstr = jnp.stack([dst0.reshape(NROW2D, CHUNK),
                      dst1.reshape(NROW2D, CHUNK)])

    h0 = jnp.pad(features, ((0, NP - N_NODES), (0, 0)))
    zrows = jnp.zeros((CHUNK, DA), jnp.float32)

    w3 = jnp.pad(W_self3, ((0, 0), (0, D - W_self3.shape[1])))
    wn3 = jnp.pad(W_neigh3, ((0, 0), (0, D - W_neigh3.shape[1])))
    b3p = jnp.pad(b3, (0, D - b3.shape[0]))

    sc_agg = _get_sc_agg()
    hn, degv = _aggregate(sc_agg, h0, src2d, dstr, zrows)
    h1 = _tc_layer(h0, hn, degv, W_self1, W_neigh1, b1.reshape(1, D),
                   relu=True)
    hn, degv = _aggregate(sc_agg, h1, src2d, dstr, zrows)
    h2 = _tc_layer(h1, hn, degv, W_self2, W_neigh2, b2.reshape(1, D),
                   relu=True)
    hn, degv = _aggregate(sc_agg, h2, src2d, dstr, zrows)
    h3 = _tc_layer(h2, hn, degv, w3, wn3, b3p.reshape(1, D), relu=False)
    return h3[:N_NODES, :W_self3.shape[1]]


# speedup vs baseline: 1.7858x; 1.0137x over previous
import functools
import jax
import jax.numpy as jnp
from jax import lax
from jax.experimental import pallas as pl
from jax.experimental.pallas import tpu as pltpu
from jax.experimental.pallas import tpu_sc as plsc

N_NODES = 10000
NP = 10240
HALF = NP // 2
SPN = 6144
D = 128
DA = D + 16
E = 320000
CHUNK = 128
NC = 2
NS = 16
T_PER_TILE = 160
GRP = 8
NROW2D = NS * T_PER_TILE
EP = NROW2D * CHUNK
ZBLK = SPN // (CHUNK * NS)


def _sc_agg_body(h_hbm, src_hbm, dst_hbm, zrows_hbm, agg_out,
                 src_idx, dst_idx, rows0, rows1, agg_sp,
                 sem0, sem1, sem_s0, sem_s1):
    c = lax.axis_index("c")
    s = lax.axis_index("s")

    pltpu.sync_copy(zrows_hbm, rows0)

    def zbody(j, carry):
        pltpu.sync_copy(rows0,
                        agg_sp.at[pl.ds((s * ZBLK + j) * CHUNK, CHUNK)])
        return carry

    lax.fori_loop(0, ZBLK, zbody, 0)

    plsc.subcore_barrier()

    def _drain_scatters(k0, k1):
        pltpu.make_async_copy(rows0, agg_sp.at[dst_idx.at[k0]],
                              sem_s0).wait()
        pltpu.make_async_copy(rows1, agg_sp.at[dst_idx.at[k1]],
                              sem_s1).wait()

    def body(g, carry):
        @pl.when(g > 0)
        def _():
            _drain_scatters(GRP - 2, GRP - 1)

        base = s * T_PER_TILE + g * GRP
        pltpu.sync_copy(src_hbm.at[pl.ds(base, GRP)], src_idx)
        pltpu.sync_copy(dst_hbm.at[c, pl.ds(base, GRP)], dst_idx)
        for k in range(0, GRP, 2):
            if k > 0:
                _drain_scatters(k - 2, k - 1)
            cp0 = pltpu.async_copy(h_hbm.at[src_idx.at[k]], rows0, sem0)
            cp1 = pltpu.async_copy(h_hbm.at[src_idx.at[k + 1]], rows1, sem1)
            cp0.wait()
            pltpu.async_copy(rows0, agg_sp.at[dst_idx.at[k]], sem_s0,
                             add=True)
            cp1.wait()
            pltpu.async_copy(rows1, agg_sp.at[dst_idx.at[k + 1]], sem_s1,
                             add=True)
        return carry

    lax.fori_loop(0, T_PER_TILE // GRP, body, 0)
    _drain_scatters(GRP - 2, GRP - 1)

    plsc.subcore_barrier()

    def obody(j, carry):
        rr = (s * ZBLK + j) * CHUNK
        pltpu.sync_copy(agg_sp.at[pl.ds(rr, CHUNK)], rows0)
        pltpu.sync_copy(rows0, agg_out.at[c, pl.ds(rr, CHUNK)])
        return carry

    lax.fori_loop(0, ZBLK, obody, 0)


@functools.cache
def _get_sc_agg():
  mesh = plsc.VectorSubcoreMesh(core_axis_name="c", subcore_axis_name="s")
  return pl.kernel(
    _sc_agg_body,
    mesh=mesh,
    out_type=[
        jax.ShapeDtypeStruct((NC, SPN, DA), jnp.float32),
    ],
    scratch_types=[
        pltpu.VMEM((GRP, CHUNK), jnp.int32),
        pltpu.VMEM((GRP, CHUNK), jnp.int32),
        pltpu.VMEM((CHUNK, DA), jnp.float32),
        pltpu.VMEM((CHUNK, DA), jnp.float32),
        pltpu.VMEM_SHARED((SPN, DA), jnp.float32),
        pltpu.SemaphoreType.DMA,
        pltpu.SemaphoreType.DMA,
        pltpu.SemaphoreType.DMA,
        pltpu.SemaphoreType.DMA,
    ],
    compiler_params=pltpu.CompilerParams(use_tc_tiling_on_sc=False),
  )


def _tc_layer_body(h_ref, hn_ref, deg_ref, ws_ref, wn_ref, b_ref, o_ref,
                   *, relu):
    rdeg = 1.0 / jnp.maximum(deg_ref[:, 0:1], 1.0)
    hn = hn_ref[:, :] * rdeg
    out = (jnp.dot(h_ref[:, :], ws_ref[:, :],
                   preferred_element_type=jnp.float32)
           + jnp.dot(hn, wn_ref[:, :], preferred_element_type=jnp.float32)
           + b_ref[:, :])
    if relu:
        out = jnp.maximum(out, 0.0)
    o_ref[:, :] = out


_BR = 1024


def _tc_layer(h, hn, degv, ws, wn, b, relu):
    return pl.pallas_call(
        functools.partial(_tc_layer_body, relu=relu),
        grid=(NP // _BR,),
        in_specs=[
            pl.BlockSpec((_BR, D), lambda i: (i, 0)),
            pl.BlockSpec((_BR, D), lambda i: (i, 0)),
            pl.BlockSpec((_BR, 8), lambda i: (i, 0)),
            pl.BlockSpec((D, D), lambda i: (0, 0)),
            pl.BlockSpec((D, D), lambda i: (0, 0)),
            pl.BlockSpec((1, D), lambda i: (0, 0)),
        ],
        out_specs=pl.BlockSpec((_BR, D), lambda i: (i, 0)),
        out_shape=jax.ShapeDtypeStruct((NP, D), jnp.float32),
    )(h, hn, degv, ws, wn, b)


def _aggregate(sc_agg, h, src2d, dstr, zrows):
    h_aug = jnp.concatenate([h, jnp.ones((NP, DA - D), jnp.float32)], axis=1)
    [agg] = sc_agg(h_aug, src2d, dstr, zrows)
    agg_full = jnp.concatenate([agg[0, :HALF], agg[1, :HALF]], axis=0)
    return agg_full[:, :D], agg_full[:, D:D + 8]


def kernel(features, edge_index, W_self1, W_neigh1, b1,
           W_self2, W_neigh2, b2, W_self3, W_neigh3, b3):
    src = edge_index[0]
    dst = edge_index[1]
    pad_e = EP - E
    src_p = jnp.concatenate([src, jnp.zeros((pad_e,), jnp.int32)])
    dst_p = jnp.concatenate([dst, jnp.full((pad_e,), N_NODES, jnp.int32)])
    src2d = src_p.reshape(NROW2D, CHUNK)
    sink = HALF + (jnp.arange(EP, dtype=jnp.int32) & (SPN - HALF - 1))
    dst0 = jnp.where(dst_p < HALF, dst_p, sink)
    dst1 = jnp.where(dst_p >= HALF, dst_p - HALF, sink)
    dstr = jnp.stack([dst0.reshape(NROW2D, CHUNK),
                      dst1.reshape(NROW2D, CHUNK)])

    h0 = jnp.pad(features, ((0, NP - N_NODES), (0, 0)))
    zrows = jnp.zeros((CHUNK, DA), jnp.float32)

    w3 = jnp.pad(W_self3, ((0, 0), (0, D - W_self3.shape[1])))
    wn3 = jnp.pad(W_neigh3, ((0, 0), (0, D - W_neigh3.shape[1])))
    b3p = jnp.pad(b3, (0, D - b3.shape[0]))

    sc_agg = _get_sc_agg()
    hn, degv = _aggregate(sc_agg, h0, src2d, dstr, zrows)
    h1 = _tc_layer(h0, hn, degv, W_self1, W_neigh1, b1.reshape(1, D),
                   relu=True)
    hn, degv = _aggregate(sc_agg, h1, src2d, dstr, zrows)
    h2 = _tc_layer(h1, hn, degv, W_self2, W_neigh2, b2.reshape(1, D),
                   relu=True)
    hn, degv = _aggregate(sc_agg, h2, src2d, dstr, zrows)
    h3 = _tc_layer(h2, hn, degv, w3, wn3, b3p.reshape(1, D), relu=False)
    return h3[:N_NODES, :W_self3.shape[1]]
